# baseline (speedup 1.0000x reference)
import functools

import jax
import jax.numpy as jnp
from jax import lax
from jax.experimental import pallas as pl
from jax.experimental.pallas import tpu as pltpu
from jax.experimental.pallas import tpu_sc as plsc

_B, _Hp, _Wp, _C = 4, 192, 192, 96
_Ho, _Wo = 384, 384
_TOTAL = _B * _Ho * _Wo * _C
_N = _B * _Hp * _Wp * _C

_NC, _NS, _L = 2, 16, 16
_W = 1 << 20
_RND = _TOTAL // _W // _NC
_NG = 4
_SH = _N // _NS

_CH1 = 4096
_NCH1 = _SH // _CH1
_NP1 = _NCH1 // 2
_FB = 512
_CH2 = 4096
_ASZ = _SH + 240 * _FB
_MB = 6144
_K2 = 2048
_UN = 4
_WR = _W // _NS
_ZB = 2048


def _body(idx_hbm, val_hbm, out_hbm, ai_hbm, av_hbm, acc, idx_st, val_st,
          ab_i0, ab_i1, ab_i2, ab_i3, ab_v0, ab_v1, ab_v2, ab_v3,
          midx, mval, zbuf, nbuf, lsem0, lsem1, fsem):
    c = lax.axis_index("c")
    s = lax.axis_index("s")

    neg1 = jnp.full((_L,), -1, jnp.int32)
    wu = jnp.full((_L,), _W, jnp.uint32)
    iota = lax.iota(jnp.int32, _L)
    ab_is = (ab_i0, ab_i1, ab_i2, ab_i3)
    ab_vs = (ab_v0, ab_v1, ab_v2, ab_v3)

    def _init(i, _):
        zbuf[pl.ds(i * _L, _L)] = jnp.zeros((_L,), jnp.float32)
        return 0
    lax.fori_loop(0, _ZB // _L, _init, 0)

    def _initn(i, _):
        nbuf[pl.ds(i * _L, _L)] = neg1
        return 0
    lax.fori_loop(0, _FB // _L, _initn, 0)

    def _initm(i, _):
        midx[pl.ds(i * _L, _L)] = neg1
        return 0
    lax.fori_loop(0, (_MB + _L) // _L, _initm, 0)

    shard0 = s * _SH
    zoff = s * _WR
    lsems = (lsem0, lsem1)
    abase = ((c * _NS + s) * _NG) * _ASZ


    def load1_descs(t, slot):
        g = shard0 + t * _CH1
        return (
            pltpu.make_async_copy(idx_hbm.at[pl.ds(g, _CH1)],
                                  idx_st.at[pl.ds(slot * _CH1, _CH1)],
                                  lsems[slot]),
            pltpu.make_async_copy(val_hbm.at[pl.ds(g, _CH1)],
                                  val_st.at[pl.ds(slot * _CH1, _CH1)],
                                  lsems[slot]),
        )

    def split_chunk(t, slot, ws):
        def vec_body(v, ps):
            for u in range(_UN):
                o = slot * _CH1 + (v * _UN + u) * _L
                i16 = idx_st[pl.ds(o, _L)]
                v16 = val_st[pl.ds(o, _L)]
                win16 = lax.shift_right_logical(i16, 20)
                keep = (win16 & 1) == c
                a16 = lax.shift_right_logical(i16, 21) & 3
                nps = []
                for a in range(_NG):
                    m = keep & (a16 == a)
                    plsc.store_compressed(ab_is[a].at[pl.ds(ps[a], _L)],
                                          i16, mask=m)
                    plsc.store_compressed(ab_vs[a].at[pl.ds(ps[a], _L)],
                                          v16, mask=m)
                    cnt = plsc.all_reduce_population_count(m)[0]
                    nps.append(ps[a] + cnt)
                ps = tuple(nps)
            return ps

        ps = lax.fori_loop(0, _CH1 // (_L * _UN), vec_body,
                           (jnp.int32(0),) * _NG)

        nws = []
        descs = []
        for a in range(_NG):
            p = ps[a]
            def pad(q, _):
                plsc.store_scatter(ab_is[a], [p + q * _L + iota], neg1)
                return 0
            rnd = ((p + _FB - 1) // _FB) * _FB
            lax.fori_loop(0, (rnd - p + _L - 1) // _L, pad, 0)
            dst0 = abase + a * _ASZ + pl.multiple_of(ws[a], _FB)
            for j in range(_CH1 // _FB):
                di = pltpu.make_async_copy(
                    ab_is[a].at[pl.ds(j * _FB, _FB)],
                    ai_hbm.at[pl.ds(dst0 + j * _FB, _FB)], fsem)
                dv = pltpu.make_async_copy(
                    ab_vs[a].at[pl.ds(j * _FB, _FB)],
                    av_hbm.at[pl.ds(dst0 + j * _FB, _FB)], fsem)
                descs.append((p > j * _FB, di, dv))
            nws.append(ws[a] + rnd)
        for cond, di, dv in descs:
            @pl.when(cond)
            def _():
                di.start()
                dv.start()
        for cond, di, dv in descs:
            @pl.when(cond)
            def _():
                di.wait()
                dv.wait()
        return tuple(nws)

    for d in load1_descs(0, 0):
        d.start()

    def pair1_body(i, ws):
        t0 = 2 * i
        for d in load1_descs(t0 + 1, 1):
            d.start()
        for d in load1_descs(t0, 0):
            d.wait()
        ws = split_chunk(t0, 0, ws)

        @pl.when(i + 1 < _NP1)
        def _():
            for d in load1_descs(t0 + 2, 0):
                d.start()
        for d in load1_descs(t0 + 1, 1):
            d.wait()
        ws = split_chunk(t0 + 1, 1, ws)
        return ws

    ws = lax.fori_loop(0, _NP1, pair1_body, (jnp.int32(0),) * _NG)

    was = []
    for a in range(_NG):
        wa = ws[a]
        gran = 2 * _CH2
        wa = pl.multiple_of(wa, _FB)
        extra = (((wa + gran - 1) // gran) * gran - wa) // _FB
        dst0 = abase + a * _ASZ + wa
        for j in range(gran // _FB - 1):
            @pl.when(extra > j)
            def _():
                pltpu.sync_copy(nbuf,
                                ai_hbm.at[pl.ds(dst0 + j * _FB, _FB)])
                pltpu.sync_copy(zbuf.at[pl.ds(0, _FB)],
                                av_hbm.at[pl.ds(dst0 + j * _FB, _FB)])
        was.append(wa + extra * _FB)


    def load2_descs(src0, t, slot):
        g = src0 + t * _CH2
        return (
            pltpu.make_async_copy(ai_hbm.at[pl.ds(g, _CH2)],
                                  idx_st.at[pl.ds(slot * _CH2, _CH2)],
                                  lsems[slot]),
            pltpu.make_async_copy(av_hbm.at[pl.ds(g, _CH2)],
                                  val_st.at[pl.ds(slot * _CH2, _CH2)],
                                  lsems[slot]),
        )

    def drain_and_wipe(fill):
        descs = []
        for j in range(_MB // _K2):
            d = pltpu.make_async_copy(
                mval.at[pl.ds(j * _K2, _K2)],
                acc.at[plsc.Indices(midx.at[pl.ds(j * _K2, _K2)],
                                    ignored_value=-1)],
                fsem)
            descs.append((fill > j * _K2, d))
        for cond, d in descs:
            @pl.when(cond)
            def _():
                d.start(add=True)
        for cond, d in descs:
            @pl.when(cond)
            def _():
                d.wait()

        def wipe(i, _):
            midx[pl.ds(i * _L, _L)] = neg1
            return 0
        lax.fori_loop(0, (fill + _L - 1) // _L, wipe, 0)

    def round_body(r, _):
        win = 2 * r + c
        base = win * _W
        a = r & 3
        wa = jnp.where(a == 0, was[0],
                       jnp.where(a == 1, was[1],
                                 jnp.where(a == 2, was[2], was[3])))
        src0 = abase + a * _ASZ
        nch = wa // _CH2

        zcps = [pltpu.make_async_copy(zbuf,
                                      acc.at[pl.ds(zoff + i * _ZB, _ZB)],
                                      lsem0)
                for i in range(_WR // _ZB)]
        for cp in zcps:
            cp.start()
        for cp in zcps:
            cp.wait()
        plsc.subcore_barrier()

        @pl.when(nch > 0)
        def _():
            for d in load2_descs(src0, 0, 0):
                d.start()

        def scan_chunk(t, slot, fill):
            def vec_body(v, fill):
                for u in range(_UN):
                    o = slot * _CH2 + (v * _UN + u) * _L
                    i16 = idx_st[pl.ds(o, _L)]
                    rel = i16 - base
                    m = plsc.bitcast(rel, jnp.uint32) < wu
                    plsc.store_compressed(midx.at[pl.ds(fill, _L)], rel,
                                          mask=m)
                    v16 = val_st[pl.ds(o, _L)]
                    plsc.store_compressed(mval.at[pl.ds(fill, _L)], v16,
                                          mask=m)
                    cnt = plsc.all_reduce_population_count(m)[0]
                    fill = fill + cnt
                return fill

            fill = lax.fori_loop(0, _CH2 // (_L * _UN), vec_body, fill)
            full = fill >= _MB - _CH2

            @pl.when(full)
            def _():
                drain_and_wipe(fill)
            return jnp.where(full, jnp.int32(0), fill)

        def pair2_body(i, fill):
            t0 = 2 * i
            for d in load2_descs(src0, t0 + 1, 1):
                d.start()
            for d in load2_descs(src0, t0, 0):
                d.wait()
            fill = scan_chunk(t0, 0, fill)

            @pl.when(t0 + 2 < nch)
            def _():
                for d in load2_descs(src0, t0 + 2, 0):
                    d.start()
            for d in load2_descs(src0, t0 + 1, 1):
                d.wait()
            fill = scan_chunk(t0 + 1, 1, fill)
            return fill

        fill = lax.fori_loop(0, nch // 2, pair2_body, jnp.int32(0))
        drain_and_wipe(fill)
        plsc.subcore_barrier()

        pltpu.sync_copy(acc.at[pl.ds(zoff, _WR)],
                        out_hbm.at[pl.ds(base + zoff, _WR)])
        plsc.subcore_barrier()
        return 0

    lax.fori_loop(0, _RND, round_body, 0)


@jax.jit
def _scatter_add(idx, val):
    mesh = plsc.VectorSubcoreMesh(core_axis_name="c", subcore_axis_name="s")
    outs = pl.kernel(
        _body,
        out_type=(
            jax.ShapeDtypeStruct((_TOTAL,), jnp.float32),
            jax.ShapeDtypeStruct((_NC * _NS * _NG * _ASZ,), jnp.int32),
            jax.ShapeDtypeStruct((_NC * _NS * _NG * _ASZ,), jnp.float32),
        ),
        mesh=mesh,
        compiler_params=pltpu.CompilerParams(
            needs_layout_passes=False, use_tc_tiling_on_sc=False),
        scratch_types=[
            pltpu.VMEM_SHARED((_W,), jnp.float32),
            pltpu.VMEM((2 * _CH1,), jnp.int32),
            pltpu.VMEM((2 * _CH1,), jnp.float32),
        ] + [pltpu.VMEM((_CH1 + _L,), jnp.int32) for _ in range(_NG)]
          + [pltpu.VMEM((_CH1 + _L,), jnp.float32) for _ in range(_NG)]
          + [
            pltpu.VMEM((_MB + _L,), jnp.int32),
            pltpu.VMEM((_MB + _L,), jnp.float32),
            pltpu.VMEM((_ZB,), jnp.float32),
            pltpu.VMEM((_FB,), jnp.int32),
            pltpu.SemaphoreType.DMA,
            pltpu.SemaphoreType.DMA,
            pltpu.SemaphoreType.DMA,
        ],
    )(idx, val)
    return outs[0]


def kernel(inputs_0, inputs_1, output_shape):
    val = inputs_0.reshape(-1)
    idx = inputs_1.reshape(-1).astype(jnp.int32)
    out = _scatter_add(idx, val)
    return out.reshape(-1, output_shape.shape[1], output_shape.shape[2],
                       inputs_0.shape[3])


# speedup vs baseline: 1.1555x; 1.1555x over previous
import functools

import jax
import jax.numpy as jnp
from jax import lax
from jax.experimental import pallas as pl
from jax.experimental.pallas import tpu as pltpu
from jax.experimental.pallas import tpu_sc as plsc

_B, _Hp, _Wp, _C = 4, 192, 192, 96
_Ho, _Wo = 384, 384
_TOTAL = _B * _Ho * _Wo * _C
_N = _B * _Hp * _Wp * _C

_NC, _NS, _L = 2, 16, 16
_W = 1 << 20
_RND = _TOTAL // _W // _NC
_NG = 4
_SH = _N // _NS

_CH1 = 3072
_NCH1 = _SH // _CH1
_NP1 = _NCH1 // 2
_FB = 512
_CH2 = 2048
_ASZ = _SH + _NCH1 * _FB + 8 * _FB
_SEG = 4096
_K2 = 2048
_UN = 4
_WR = _W // _NS
_ZB = 2048


def _body(idx_hbm, val_hbm, out_hbm, ai_hbm, av_hbm, acc, idx_st, val_st,
          ab_i0, ab_i1, ab_i2, ab_i3, ab_v0, ab_v1, ab_v2, ab_v3,
          midx, mval, zbuf, nbuf, lsem0, lsem1, fsem):
    c = lax.axis_index("c")
    s = lax.axis_index("s")

    neg1 = jnp.full((_L,), -1, jnp.int32)
    wu = jnp.full((_L,), _W, jnp.uint32)
    iota = lax.iota(jnp.int32, _L)
    ab_is = (ab_i0, ab_i1, ab_i2, ab_i3)
    ab_vs = (ab_v0, ab_v1, ab_v2, ab_v3)

    def _init(i, _):
        zbuf[pl.ds(i * _L, _L)] = jnp.zeros((_L,), jnp.float32)
        return 0
    lax.fori_loop(0, _ZB // _L, _init, 0)

    def _initn(i, _):
        nbuf[pl.ds(i * _L, _L)] = neg1
        return 0
    lax.fori_loop(0, _FB // _L, _initn, 0)

    def _initm(i, _):
        midx[pl.ds(i * _L, _L)] = neg1
        return 0
    lax.fori_loop(0, 2 * (_SEG + _L) // _L, _initm, 0)

    shard0 = s * _SH
    zoff = s * _WR
    lsems = (lsem0, lsem1)
    abase = ((c * _NS + s) * _NG) * _ASZ


    def load1_descs(t, slot):
        g = shard0 + t * _CH1
        return (
            pltpu.make_async_copy(idx_hbm.at[pl.ds(g, _CH1)],
                                  idx_st.at[pl.ds(slot * _CH1, _CH1)],
                                  lsems[slot]),
            pltpu.make_async_copy(val_hbm.at[pl.ds(g, _CH1)],
                                  val_st.at[pl.ds(slot * _CH1, _CH1)],
                                  lsems[slot]),
        )

    def split_chunk(t, slot, ws):
        def vec_body(v, ps):
            for u in range(_UN):
                o = slot * _CH1 + (v * _UN + u) * _L
                i16 = idx_st[pl.ds(o, _L)]
                v16 = val_st[pl.ds(o, _L)]
                win16 = lax.shift_right_logical(i16, 20)
                keep = (win16 & 1) == c
                a16 = lax.shift_right_logical(i16, 21) & 3
                nps = []
                for a in range(_NG):
                    m = keep & (a16 == a)
                    plsc.store_compressed(ab_is[a].at[pl.ds(ps[a], _L)],
                                          i16, mask=m)
                    plsc.store_compressed(ab_vs[a].at[pl.ds(ps[a], _L)],
                                          v16, mask=m)
                    cnt = plsc.all_reduce_population_count(m)[0]
                    nps.append(ps[a] + cnt)
                ps = tuple(nps)
            return ps

        ps = lax.fori_loop(0, _CH1 // (_L * _UN), vec_body,
                           (jnp.int32(0),) * _NG)

        nws = []
        for a in range(_NG):
            p = ps[a]
            def pad(q, _):
                plsc.store_scatter(ab_is[a], [p + q * _L + iota], neg1)
                return 0
            rnd = ((p + _FB - 1) // _FB) * _FB
            lax.fori_loop(0, (rnd - p + _L - 1) // _L, pad, 0)
            dst0 = abase + a * _ASZ + pl.multiple_of(ws[a], _FB)
            for j in range(_CH1 // _FB):
                @pl.when(p > j * _FB)
                def _():
                    pltpu.sync_copy(
                        ab_is[a].at[pl.ds(j * _FB, _FB)],
                        ai_hbm.at[pl.ds(dst0 + j * _FB, _FB)])
                    pltpu.sync_copy(
                        ab_vs[a].at[pl.ds(j * _FB, _FB)],
                        av_hbm.at[pl.ds(dst0 + j * _FB, _FB)])
            nws.append(ws[a] + rnd)
        return tuple(nws)

    for d in load1_descs(0, 0):
        d.start()

    def pair1_body(i, ws):
        t0 = 2 * i
        for d in load1_descs(t0 + 1, 1):
            d.start()
        for d in load1_descs(t0, 0):
            d.wait()
        ws = split_chunk(t0, 0, ws)

        @pl.when(i + 1 < _NP1)
        def _():
            for d in load1_descs(t0 + 2, 0):
                d.start()
        for d in load1_descs(t0 + 1, 1):
            d.wait()
        ws = split_chunk(t0 + 1, 1, ws)
        return ws

    ws = lax.fori_loop(0, _NP1, pair1_body, (jnp.int32(0),) * _NG)

    was = []
    for a in range(_NG):
        wa = ws[a]
        gran = 2 * _CH2
        wa = pl.multiple_of(wa, _FB)
        extra = (((wa + gran - 1) // gran) * gran - wa) // _FB
        dst0 = abase + a * _ASZ + wa
        for j in range(gran // _FB - 1):
            @pl.when(extra > j)
            def _():
                pltpu.sync_copy(nbuf,
                                ai_hbm.at[pl.ds(dst0 + j * _FB, _FB)])
                pltpu.sync_copy(zbuf.at[pl.ds(0, _FB)],
                                av_hbm.at[pl.ds(dst0 + j * _FB, _FB)])
        was.append(wa + extra * _FB)


    def load2_descs(src0, t, slot):
        g = src0 + t * _CH2
        return (
            pltpu.make_async_copy(ai_hbm.at[pl.ds(g, _CH2)],
                                  idx_st.at[pl.ds(slot * _CH2, _CH2)],
                                  lsems[slot]),
            pltpu.make_async_copy(av_hbm.at[pl.ds(g, _CH2)],
                                  val_st.at[pl.ds(slot * _CH2, _CH2)],
                                  lsems[slot]),
        )

    def drain_and_wipe(fills):
        descs = []
        for seg in range(2):
            s0 = seg * (_SEG + _L)
            for j in range(_SEG // _K2):
                d = pltpu.make_async_copy(
                    mval.at[pl.ds(s0 + j * _K2, _K2)],
                    acc.at[plsc.Indices(
                        midx.at[pl.ds(s0 + j * _K2, _K2)],
                        ignored_value=-1)],
                    fsem)
                descs.append((fills[seg] > j * _K2, d))
        for cond, d in descs:
            @pl.when(cond)
            def _():
                d.start(add=True)
        for cond, d in descs:
            @pl.when(cond)
            def _():
                d.wait()

        for seg in range(2):
            s0 = seg * (_SEG + _L)

            def wipe(i, _):
                midx[pl.ds(s0 + i * _L, _L)] = neg1
                return 0
            lax.fori_loop(0, (fills[seg] + _L - 1) // _L, wipe, 0)

    def round_body(r, _):
        win = 2 * r + c
        base = win * _W
        a = r & 3
        wa = jnp.where(a == 0, was[0],
                       jnp.where(a == 1, was[1],
                                 jnp.where(a == 2, was[2], was[3])))
        src0 = abase + a * _ASZ
        nch = wa // _CH2

        zcps = [pltpu.make_async_copy(zbuf,
                                      acc.at[pl.ds(zoff + i * _ZB, _ZB)],
                                      lsem0)
                for i in range(_WR // _ZB)]
        for cp in zcps:
            cp.start()
        for cp in zcps:
            cp.wait()
        plsc.subcore_barrier()

        @pl.when(nch > 0)
        def _():
            for d in load2_descs(src0, 0, 0):
                d.start()

        def scan_chunk(t, slot, fills):
            def vec_body(v, fills):
                f = list(fills)
                for u in range(_UN):
                    seg = u & 1
                    s0 = seg * (_SEG + _L)
                    o = slot * _CH2 + (v * _UN + u) * _L
                    i16 = idx_st[pl.ds(o, _L)]
                    rel = i16 - base
                    m = plsc.bitcast(rel, jnp.uint32) < wu
                    plsc.store_compressed(midx.at[pl.ds(s0 + f[seg], _L)],
                                          rel, mask=m)
                    v16 = val_st[pl.ds(o, _L)]
                    plsc.store_compressed(mval.at[pl.ds(s0 + f[seg], _L)],
                                          v16, mask=m)
                    cnt = plsc.all_reduce_population_count(m)[0]
                    f[seg] = f[seg] + cnt
                return tuple(f)

            fills = lax.fori_loop(0, _CH2 // (_L * _UN), vec_body, fills)
            full = ((fills[0] >= _SEG - _CH2 // 2) |
                    (fills[1] >= _SEG - _CH2 // 2))

            @pl.when(full)
            def _():
                drain_and_wipe(fills)
            z = jnp.int32(0)
            return (jnp.where(full, z, fills[0]),
                    jnp.where(full, z, fills[1]))

        def pair2_body(i, fills):
            t0 = 2 * i
            for d in load2_descs(src0, t0 + 1, 1):
                d.start()
            for d in load2_descs(src0, t0, 0):
                d.wait()
            fills = scan_chunk(t0, 0, fills)

            @pl.when(t0 + 2 < nch)
            def _():
                for d in load2_descs(src0, t0 + 2, 0):
                    d.start()
            for d in load2_descs(src0, t0 + 1, 1):
                d.wait()
            fills = scan_chunk(t0 + 1, 1, fills)
            return fills

        fills = lax.fori_loop(0, nch // 2, pair2_body,
                              (jnp.int32(0), jnp.int32(0)))
        drain_and_wipe(fills)
        plsc.subcore_barrier()

        pltpu.sync_copy(acc.at[pl.ds(zoff, _WR)],
                        out_hbm.at[pl.ds(base + zoff, _WR)])
        plsc.subcore_barrier()
        return 0

    lax.fori_loop(0, _RND, round_body, 0)


@jax.jit
def _scatter_add(idx, val):
    mesh = plsc.VectorSubcoreMesh(core_axis_name="c", subcore_axis_name="s")
    outs = pl.kernel(
        _body,
        out_type=(
            jax.ShapeDtypeStruct((_TOTAL,), jnp.float32),
            jax.ShapeDtypeStruct((_NC * _NS * _NG * _ASZ,), jnp.int32),
            jax.ShapeDtypeStruct((_NC * _NS * _NG * _ASZ,), jnp.float32),
        ),
        mesh=mesh,
        compiler_params=pltpu.CompilerParams(
            needs_layout_passes=False, use_tc_tiling_on_sc=False),
        scratch_types=[
            pltpu.VMEM_SHARED((_W,), jnp.float32),
            pltpu.VMEM((2 * _CH1,), jnp.int32),
            pltpu.VMEM((2 * _CH1,), jnp.float32),
        ] + [pltpu.VMEM((_CH1 + _L,), jnp.int32) for _ in range(_NG)]
          + [pltpu.VMEM((_CH1 + _L,), jnp.float32) for _ in range(_NG)]
          + [
            pltpu.VMEM((2 * (_SEG + _L),), jnp.int32),
            pltpu.VMEM((2 * (_SEG + _L),), jnp.float32),
            pltpu.VMEM((_ZB,), jnp.float32),
            pltpu.VMEM((_FB,), jnp.int32),
            pltpu.SemaphoreType.DMA,
            pltpu.SemaphoreType.DMA,
            pltpu.SemaphoreType.DMA,
        ],
    )(idx, val)
    return outs[0]


def kernel(inputs_0, inputs_1, output_shape):
    val = inputs_0.reshape(-1)
    idx = inputs_1.reshape(-1).astype(jnp.int32)
    out = _scatter_add(idx, val)
    return out.reshape(-1, output_shape.shape[1], output_shape.shape[2],
                       inputs_0.shape[3])
